# baseline (speedup 1.0000x reference)
import functools

import jax
import jax.numpy as jnp
from jax.experimental import pallas as pl
from jax.experimental.pallas import tpu as pltpu

_EPS = 1e-6
_CAP_FACTOR = 1.25


def _gate_kernel(x_ref, w_ref, b_ref, o_ref, acc_ref, *, capacity, kb):
    batch, sblk, kchunk = x_ref.shape
    e = w_ref.shape[0]
    k = pl.program_id(1)
    x = x_ref[...].reshape(batch * sblk, kchunk)
    part = jax.lax.dot_general(
        w_ref[...], x, (((1,), (1,)), ((), ())),
        preferred_element_type=jnp.float32)

    @pl.when(k == 0)
    def _():
        acc_ref[...] = part

    @pl.when((k > 0) & (k < kb - 1))
    def _():
        acc_ref[...] = acc_ref[...] + part

    @pl.when(k == kb - 1)
    def _():
        logits = acc_ref[...] + part + jnp.transpose(b_ref[...], (1, 0))

        m = jnp.max(logits, axis=0, keepdims=True)
        ex = jnp.exp(logits - m)
        probs = ex / jnp.sum(ex, axis=0, keepdims=True)

        iota = jax.lax.broadcasted_iota(jnp.int32, logits.shape, 0)
        i1 = jnp.min(jnp.where(logits == m, iota, e), axis=0, keepdims=True)
        mask1 = iota == i1
        neg = jnp.float32(-jnp.inf)
        l2 = jnp.where(mask1, neg, logits)
        m2 = jnp.max(l2, axis=0, keepdims=True)
        i2 = jnp.min(jnp.where(l2 == m2, iota, e), axis=0, keepdims=True)
        mask = mask1 | (iota == i2)

        masked = jnp.where(mask, probs, jnp.float32(0.0))
        den = jnp.float32(_EPS)
        for b in range(batch):
            den = den + masked[:, b * sblk:(b + 1) * sblk]
        scale = jnp.float32(capacity) / den
        for b in range(batch):
            o_ref[b] = masked[:, b * sblk:(b + 1) * sblk] * scale


def kernel(X, Wg, bg):
    batch, seq, dim = X.shape
    e = Wg.shape[0]
    capacity = int(_CAP_FACTOR * batch)
    sblk = 256
    kb = 2
    kchunk = dim // kb
    grid = (seq // sblk, kb)
    out = pl.pallas_call(
        functools.partial(_gate_kernel, capacity=capacity, kb=kb),
        grid=grid,
        in_specs=[
            pl.BlockSpec((batch, sblk, kchunk), lambda i, k: (0, i, k)),
            pl.BlockSpec((e, kchunk), lambda i, k: (0, k)),
            pl.BlockSpec((1, e), lambda i, k: (0, 0)),
        ],
        out_specs=pl.BlockSpec((batch, e, sblk), lambda i, k: (0, 0, i)),
        out_shape=jax.ShapeDtypeStruct((batch, e, seq), jnp.float32),
        scratch_shapes=[pltpu.VMEM((e, batch * sblk), jnp.float32)],
    )(X, Wg, bg.reshape(1, e))
    return (jnp.transpose(out, (0, 2, 1)), None)


# speedup vs baseline: 1.0433x; 1.0433x over previous
import functools

import jax
import jax.numpy as jnp
from jax.experimental import pallas as pl

_EPS = 1e-6
_CAP_FACTOR = 1.25


def _masked_softmax(logits, e):
    m = jnp.max(logits, axis=0, keepdims=True)
    ex = jnp.exp(logits - m)
    probs = ex / jnp.sum(ex, axis=0, keepdims=True)

    iota = jax.lax.broadcasted_iota(jnp.int32, logits.shape, 0)
    i1 = jnp.min(jnp.where(logits == m, iota, e), axis=0, keepdims=True)
    mask1 = iota == i1
    l2 = jnp.where(mask1, jnp.float32(-jnp.inf), logits)
    m2 = jnp.max(l2, axis=0, keepdims=True)
    i2 = jnp.min(jnp.where(l2 == m2, iota, e), axis=0, keepdims=True)
    mask = mask1 | (iota == i2)
    return jnp.where(mask, probs, jnp.float32(0.0))


def _gate_kernel(x_ref, w_ref, b_ref, o_ref, *, capacity, batch):
    _, sblk, dim = x_ref.shape
    e = w_ref.shape[0]
    b = pl.program_id(1)
    x = x_ref[...].reshape(sblk, dim)
    logits = jax.lax.dot_general(
        w_ref[...], x, (((1,), (1,)), ((), ())),
        preferred_element_type=jnp.float32)
    logits = logits + jnp.transpose(b_ref[...], (1, 0))
    masked = _masked_softmax(logits, e)

    for bi in range(batch - 1):
        @pl.when(b == bi)
        def _(bi=bi):
            o_ref[bi] = masked

    @pl.when(b == batch - 1)
    def _():
        prev = [o_ref[bi] for bi in range(batch - 1)]
        den = masked + jnp.float32(_EPS)
        for p in prev:
            den = den + p
        scale = jnp.float32(capacity) / den
        for bi in range(batch - 1):
            o_ref[bi] = prev[bi] * scale
        o_ref[batch - 1] = masked * scale


def kernel(X, Wg, bg):
    batch, seq, dim = X.shape
    e = Wg.shape[0]
    capacity = int(_CAP_FACTOR * batch)
    sblk = 512
    grid = (seq // sblk, batch)
    out = pl.pallas_call(
        functools.partial(_gate_kernel, capacity=capacity, batch=batch),
        grid=grid,
        in_specs=[
            pl.BlockSpec((1, sblk, dim), lambda i, b: (b, i, 0)),
            pl.BlockSpec((e, dim), lambda i, b: (0, 0)),
            pl.BlockSpec((1, e), lambda i, b: (0, 0)),
        ],
        out_specs=pl.BlockSpec((batch, e, sblk), lambda i, b: (0, 0, i)),
        out_shape=jax.ShapeDtypeStruct((batch, e, seq), jnp.float32),
    )(X, Wg, bg.reshape(1, e))
    return (jnp.transpose(out, (0, 2, 1)), None)


# speedup vs baseline: 1.0847x; 1.0397x over previous
import functools

import jax
import jax.numpy as jnp
from jax.experimental import pallas as pl

_EPS = 1e-6
_CAP_FACTOR = 1.25


def _masked_softmax(logits, e):
    m = jnp.max(logits, axis=0, keepdims=True)
    ex = jnp.exp(logits - m)
    probs = ex / jnp.sum(ex, axis=0, keepdims=True)

    iota = jax.lax.broadcasted_iota(jnp.int32, logits.shape, 0)
    i1 = jnp.min(jnp.where(logits == m, iota, e), axis=0, keepdims=True)
    mask1 = iota == i1
    l2 = jnp.where(mask1, jnp.float32(-jnp.inf), logits)
    m2 = jnp.max(l2, axis=0, keepdims=True)
    i2 = jnp.min(jnp.where(l2 == m2, iota, e), axis=0, keepdims=True)
    mask = mask1 | (iota == i2)
    return jnp.where(mask, probs, jnp.float32(0.0))


def _gate_kernel(x_ref, w_ref, b_ref, o_ref, *, capacity, batch):
    _, sblk, dim = x_ref.shape
    e = w_ref.shape[0]
    b = pl.program_id(1)
    x = x_ref[...].reshape(sblk, dim)
    logits = jax.lax.dot_general(
        w_ref[...], x, (((1,), (1,)), ((), ())),
        preferred_element_type=jnp.float32)
    logits = logits + jnp.transpose(b_ref[...], (1, 0))
    masked = _masked_softmax(logits, e)

    for bi in range(batch - 1):
        @pl.when(b == bi)
        def _(bi=bi):
            o_ref[bi] = masked

    @pl.when(b == batch - 1)
    def _():
        prev = [o_ref[bi] for bi in range(batch - 1)]
        den = masked + jnp.float32(_EPS)
        for p in prev:
            den = den + p
        scale = jnp.float32(capacity) / den
        for bi in range(batch - 1):
            o_ref[bi] = prev[bi] * scale
        o_ref[batch - 1] = masked * scale


def kernel(X, Wg, bg):
    batch, seq, dim = X.shape
    e = Wg.shape[0]
    capacity = int(_CAP_FACTOR * batch)
    sblk = 1024
    grid = (seq // sblk, batch)
    out = pl.pallas_call(
        functools.partial(_gate_kernel, capacity=capacity, batch=batch),
        grid=grid,
        in_specs=[
            pl.BlockSpec((1, sblk, dim), lambda i, b: (b, i, 0)),
            pl.BlockSpec((e, dim), lambda i, b: (0, 0)),
            pl.BlockSpec((1, e), lambda i, b: (0, 0)),
        ],
        out_specs=pl.BlockSpec((batch, e, sblk), lambda i, b: (0, 0, i)),
        out_shape=jax.ShapeDtypeStruct((batch, e, seq), jnp.float32),
    )(X, Wg, bg.reshape(1, e))
    return (jnp.transpose(out, (0, 2, 1)), None)
